# Initial kernel scaffold; baseline (speedup 1.0000x reference)
import jax
import jax.numpy as jnp
from jax.experimental import pallas as pl


def kernel(x, edge_index, params):
    raise NotImplementedError("write your pallas kernel here")


# speedup vs baseline: 2.2469x; 2.2469x over previous
import functools

import jax
import jax.numpy as jnp
from jax import lax
from jax.experimental import pallas as pl
from jax.experimental.pallas import tpu as pltpu

N_NODES = 10000
N_PAD = 10240
D = 256
H = 8
C = 256
HC = H * C
NTAB = 32
RB = 256
GRID = N_PAD // RB
INV_SQRT_C = 1.0 / 16.0


def _ln(y, g, b):
    m = jnp.mean(y, axis=-1, keepdims=True)
    v = jnp.mean((y - m) ** 2, axis=-1, keepdims=True)
    return (y - m) * lax.rsqrt(v + 1e-5) * g + b


def _gelu(x):
    return 0.5 * x * (1.0 + lax.erf(x * (2.0 ** -0.5)))


def _embed_body(ids_ref, emb_ref, out_ref):
    ids = ids_ref[...]
    emb = emb_ref[...]
    acc = jnp.zeros((RB, D), jnp.float32)
    for j in range(3):
        oh = (ids[:, j][:, None]
              == lax.broadcasted_iota(jnp.int32, (RB, NTAB), 1)).astype(jnp.float32)
        acc = acc + jnp.dot(oh, emb, preferred_element_type=jnp.float32)
    out_ref[...] = acc


def _embed(ids_pad, emb_pad):
    return pl.pallas_call(
        _embed_body,
        grid=(GRID,),
        in_specs=[
            pl.BlockSpec((RB, 8), lambda i: (i, 0)),
            pl.BlockSpec((NTAB, D), lambda i: (0, 0)),
        ],
        out_specs=pl.BlockSpec((RB, D), lambda i: (i, 0)),
        out_shape=jax.ShapeDtypeStruct((N_PAD, D), jnp.float32),
    )(ids_pad, emb_pad)


def _preattn_body(h_ref, w_ref, b_ref, g_ref, q_ref, k_ref, v_ref, s_ref):
    gb = g_ref[...]
    y = _ln(h_ref[...], gb[0:1, :], gb[1:2, :])
    w = w_ref[...]
    b = b_ref[0:1, :]
    o = jnp.dot(y, w, preferred_element_type=jnp.float32) + b
    q_ref[...] = o[:, :HC]
    k_ref[...] = o[:, HC:2 * HC]
    v_ref[...] = o[:, 2 * HC:3 * HC]
    s_ref[...] = o[:, 3 * HC:]


def _preattn(h, wcat, bcat, gb):
    TOT = 3 * HC + D
    return pl.pallas_call(
        _preattn_body,
        grid=(GRID,),
        in_specs=[
            pl.BlockSpec((RB, D), lambda i: (i, 0)),
            pl.BlockSpec((D, TOT), lambda i: (0, 0)),
            pl.BlockSpec((8, TOT), lambda i: (0, 0)),
            pl.BlockSpec((8, D), lambda i: (0, 0)),
        ],
        out_specs=[
            pl.BlockSpec((RB, HC), lambda i: (i, 0)),
            pl.BlockSpec((RB, HC), lambda i: (i, 0)),
            pl.BlockSpec((RB, HC), lambda i: (i, 0)),
            pl.BlockSpec((RB, D), lambda i: (i, 0)),
        ],
        out_shape=[
            jax.ShapeDtypeStruct((N_PAD, HC), jnp.float32),
            jax.ShapeDtypeStruct((N_PAD, HC), jnp.float32),
            jax.ShapeDtypeStruct((N_PAD, HC), jnp.float32),
            jax.ShapeDtypeStruct((N_PAD, D), jnp.float32),
        ],
    )(h, wcat, bcat, gb)


def _post_body(h_ref, num_ref, den_ref, skip_ref, w1_ref, w2_ref, b_ref,
               g_ref, out_ref):
    num = num_ref[...]
    den = den_ref[...]
    acc = jnp.zeros((RB, D), jnp.float32)
    for hh in range(H):
        acc = acc + num[:, hh * C:(hh + 1) * C] / (den[:, hh:hh + 1] + 1e-16)
    h1 = h_ref[...] + acc * (1.0 / H) + skip_ref[...]
    gb = g_ref[...]
    y = _ln(h1, gb[0:1, :], gb[1:2, :])
    b = b_ref[...]
    t = _gelu(jnp.dot(y, w1_ref[...], preferred_element_type=jnp.float32)
              + b[0:1, :])
    z = jnp.dot(t, w2_ref[...], preferred_element_type=jnp.float32) + b[1:2, :]
    out_ref[...] = h1 + z


def _post(h, numer, den, skip, w1, w2, bcat, gb):
    return pl.pallas_call(
        _post_body,
        grid=(GRID,),
        in_specs=[
            pl.BlockSpec((RB, D), lambda i: (i, 0)),
            pl.BlockSpec((RB, HC), lambda i: (i, 0)),
            pl.BlockSpec((RB, 16), lambda i: (i, 0)),
            pl.BlockSpec((RB, D), lambda i: (i, 0)),
            pl.BlockSpec((D, D), lambda i: (0, 0)),
            pl.BlockSpec((D, D), lambda i: (0, 0)),
            pl.BlockSpec((8, D), lambda i: (0, 0)),
            pl.BlockSpec((8, D), lambda i: (0, 0)),
        ],
        out_specs=pl.BlockSpec((RB, D), lambda i: (i, 0)),
        out_shape=jax.ShapeDtypeStruct((N_PAD, D), jnp.float32),
    )(h, numer, den, skip, w1, w2, bcat, gb)


def _final_body(h_ref, wm_ref, gbw_ref, out_ref):
    gbw = gbw_ref[...]
    y = _ln(h_ref[...], gbw[0:1, :], gbw[1:2, :])
    wm = wm_ref[...]
    for j in range(3):
        y = _gelu(jnp.dot(y, wm[j * D:(j + 1) * D, :],
                          preferred_element_type=jnp.float32))
    sc = jnp.sum(y * gbw[2:3, :], axis=1)
    rid = pl.program_id(0) * RB + lax.broadcasted_iota(jnp.int32, (RB,), 0)
    sc = jnp.where(rid < N_NODES, sc, 0.0)
    out_ref[...] = jnp.full((8, 128), jnp.sum(sc), jnp.float32)


def _final(h, wm, gbw):
    return pl.pallas_call(
        _final_body,
        grid=(GRID,),
        in_specs=[
            pl.BlockSpec((RB, D), lambda i: (i, 0)),
            pl.BlockSpec((3 * D, D), lambda i: (0, 0)),
            pl.BlockSpec((8, D), lambda i: (0, 0)),
        ],
        out_specs=pl.BlockSpec((8, 128), lambda i: (i, 0)),
        out_shape=jax.ShapeDtypeStruct((GRID * 8, 128), jnp.float32),
    )(h, wm, gbw)


def _edge_phase_jnp(q, k, v, src_s, dst_s):
    alpha = jnp.sum(
        (q[dst_s] * k[src_s]).reshape(-1, H, C), axis=-1) * INV_SQRT_C
    ex = jnp.exp(alpha)
    den = jax.ops.segment_sum(ex, dst_s, num_segments=N_PAD)
    numer = jax.ops.segment_sum(
        (v[src_s].reshape(-1, H, C) * ex[:, :, None]).reshape(-1, HC),
        dst_s, num_segments=N_PAD)
    den16 = jnp.pad(den, ((0, 0), (0, 8)))
    return numer, den16


def kernel(x, edge_index, params):
    ids = x[:, 0, :].astype(jnp.int32)
    ids_pad = jnp.pad(ids, ((0, N_PAD - N_NODES), (0, 5)),
                      constant_values=NTAB - 1)
    emb_pad = jnp.zeros((NTAB, D), jnp.float32).at[:25].set(params["node_emb"])

    src = edge_index[0]
    dst = edge_index[1]
    order = jnp.argsort(dst)
    src_s = src[order].astype(jnp.int32)
    dst_s = dst[order].astype(jnp.int32)

    h = _embed(ids_pad, emb_pad)

    for lp in params["layers"]:
        wcat = jnp.concatenate(
            [lp["Wq"].T, lp["Wk"].T, lp["Wv"].T, lp["Wskip"].T], axis=1)
        bcat = jnp.zeros((8, 3 * HC + D), jnp.float32).at[0].set(
            jnp.concatenate([lp["bq"], lp["bk"], lp["bv"], lp["bskip"]]))
        gb1 = jnp.zeros((8, D), jnp.float32).at[0].set(lp["ln1_g"]).at[1].set(
            lp["ln1_b"])
        q, k, v, skip = _preattn(h, wcat, bcat, gb1)

        numer, den16 = _edge_phase_jnp(q, k, v, src_s, dst_s)

        gb2 = jnp.zeros((8, D), jnp.float32).at[0].set(lp["ln2_g"]).at[1].set(
            lp["ln2_b"])
        bff = jnp.zeros((8, D), jnp.float32).at[0].set(lp["b1"]).at[1].set(
            lp["b2"])
        h = _post(h, numer, den16, skip, lp["W1"].T, lp["W2"].T, bff, gb2)

    wm = jnp.concatenate([w.T for (w, _) in params["mlp"]], axis=0)
    gbw = jnp.zeros((8, D), jnp.float32).at[0].set(params["fln_g"]).at[1].set(
        params["fln_b"]).at[2].set(params["Wout"][0])
    parts = _final(h, wm, gbw)
    total = jnp.sum(parts[::8, 0])
    return total / N_NODES + params["bout"][0]


# speedup vs baseline: 2.6804x; 1.1929x over previous
import functools

import jax
import jax.numpy as jnp
from jax import lax
from jax.experimental import pallas as pl
from jax.experimental.pallas import tpu as pltpu
from jax.experimental.pallas import tpu_sc as plsc

N_NODES = 10000
N_PAD = 10240
D = 256
H = 8
C = 256
HC = H * C
NTAB = 32
RB = 256
GRID = N_PAD // RB
INV_SQRT_C = 1.0 / 16.0


def _ln(y, g, b):
    m = jnp.mean(y, axis=-1, keepdims=True)
    v = jnp.mean((y - m) ** 2, axis=-1, keepdims=True)
    return (y - m) * lax.rsqrt(v + 1e-5) * g + b


def _gelu(x):
    return 0.5 * x * (1.0 + lax.erf(x * (2.0 ** -0.5)))


def _embed_body(ids_ref, emb_ref, out_ref):
    ids = ids_ref[...]
    emb = emb_ref[...]
    acc = jnp.zeros((RB, D), jnp.float32)
    for j in range(3):
        oh = (ids[:, j][:, None]
              == lax.broadcasted_iota(jnp.int32, (RB, NTAB), 1)).astype(jnp.float32)
        acc = acc + jnp.dot(oh, emb, preferred_element_type=jnp.float32)
    out_ref[...] = acc


def _embed(ids_pad, emb_pad):
    return pl.pallas_call(
        _embed_body,
        grid=(GRID,),
        in_specs=[
            pl.BlockSpec((RB, 8), lambda i: (i, 0)),
            pl.BlockSpec((NTAB, D), lambda i: (0, 0)),
        ],
        out_specs=pl.BlockSpec((RB, D), lambda i: (i, 0)),
        out_shape=jax.ShapeDtypeStruct((N_PAD, D), jnp.float32),
    )(ids_pad, emb_pad)


def _preattn_body(h_ref, w_ref, b_ref, g_ref, q_ref, k_ref, v_ref, s_ref):
    gb = g_ref[...]
    y = _ln(h_ref[...], gb[0:1, :], gb[1:2, :])
    w = w_ref[...]
    b = b_ref[0:1, :]
    o = jnp.dot(y, w, preferred_element_type=jnp.float32) + b
    q_ref[...] = o[:, :HC]
    k_ref[...] = o[:, HC:2 * HC]
    v_ref[...] = o[:, 2 * HC:3 * HC]
    s_ref[...] = o[:, 3 * HC:]


def _preattn(h, wcat, bcat, gb):
    TOT = 3 * HC + D
    return pl.pallas_call(
        _preattn_body,
        grid=(GRID,),
        in_specs=[
            pl.BlockSpec((RB, D), lambda i: (i, 0)),
            pl.BlockSpec((D, TOT), lambda i: (0, 0)),
            pl.BlockSpec((8, TOT), lambda i: (0, 0)),
            pl.BlockSpec((8, D), lambda i: (0, 0)),
        ],
        out_specs=[
            pl.BlockSpec((RB, HC), lambda i: (i, 0)),
            pl.BlockSpec((RB, HC), lambda i: (i, 0)),
            pl.BlockSpec((RB, HC), lambda i: (i, 0)),
            pl.BlockSpec((RB, D), lambda i: (i, 0)),
        ],
        out_shape=[
            jax.ShapeDtypeStruct((N_PAD, HC), jnp.float32),
            jax.ShapeDtypeStruct((N_PAD, HC), jnp.float32),
            jax.ShapeDtypeStruct((N_PAD, HC), jnp.float32),
            jax.ShapeDtypeStruct((N_PAD, D), jnp.float32),
        ],
    )(h, wcat, bcat, gb)


def _post_body(h_ref, num_ref, den_ref, skip_ref, w1_ref, w2_ref, b_ref,
               g_ref, out_ref):
    num = num_ref[...]
    den = den_ref[...]
    acc = jnp.zeros((RB, D), jnp.float32)
    for hh in range(H):
        acc = acc + num[:, hh * C:(hh + 1) * C] / (den[:, hh:hh + 1] + 1e-16)
    h1 = h_ref[...] + acc * (1.0 / H) + skip_ref[...]
    gb = g_ref[...]
    y = _ln(h1, gb[0:1, :], gb[1:2, :])
    b = b_ref[...]
    t = _gelu(jnp.dot(y, w1_ref[...], preferred_element_type=jnp.float32)
              + b[0:1, :])
    z = jnp.dot(t, w2_ref[...], preferred_element_type=jnp.float32) + b[1:2, :]
    out_ref[...] = h1 + z


def _post(h, numer, den, skip, w1, w2, bcat, gb):
    return pl.pallas_call(
        _post_body,
        grid=(GRID,),
        in_specs=[
            pl.BlockSpec((RB, D), lambda i: (i, 0)),
            pl.BlockSpec((RB, HC), lambda i: (i, 0)),
            pl.BlockSpec((RB, 16), lambda i: (i, 0)),
            pl.BlockSpec((RB, D), lambda i: (i, 0)),
            pl.BlockSpec((D, D), lambda i: (0, 0)),
            pl.BlockSpec((D, D), lambda i: (0, 0)),
            pl.BlockSpec((8, D), lambda i: (0, 0)),
            pl.BlockSpec((8, D), lambda i: (0, 0)),
        ],
        out_specs=pl.BlockSpec((RB, D), lambda i: (i, 0)),
        out_shape=jax.ShapeDtypeStruct((N_PAD, D), jnp.float32),
    )(h, numer, den, skip, w1, w2, bcat, gb)


def _final_body(h_ref, wm_ref, gbw_ref, out_ref):
    gbw = gbw_ref[...]
    y = _ln(h_ref[...], gbw[0:1, :], gbw[1:2, :])
    wm = wm_ref[...]
    for j in range(3):
        y = _gelu(jnp.dot(y, wm[j * D:(j + 1) * D, :],
                          preferred_element_type=jnp.float32))
    sc = jnp.sum(y * gbw[2:3, :], axis=1)
    rid = pl.program_id(0) * RB + lax.broadcasted_iota(jnp.int32, (RB,), 0)
    sc = jnp.where(rid < N_NODES, sc, 0.0)
    out_ref[...] = jnp.full((8, 128), jnp.sum(sc), jnp.float32)


def _final(h, wm, gbw):
    return pl.pallas_call(
        _final_body,
        grid=(GRID,),
        in_specs=[
            pl.BlockSpec((RB, D), lambda i: (i, 0)),
            pl.BlockSpec((3 * D, D), lambda i: (0, 0)),
            pl.BlockSpec((8, D), lambda i: (0, 0)),
        ],
        out_specs=pl.BlockSpec((8, 128), lambda i: (i, 0)),
        out_shape=jax.ShapeDtypeStruct((GRID * 8, 128), jnp.float32),
    )(h, wm, gbw)


NPW = N_PAD // 32
ECAP = 8192


def _edge_sc_body(q_hbm, k_hbm, v_hbm, srcp_hbm, degw_hbm, numer_hbm, den_hbm,
                  src_v, deg_v, qrow, kbuf, vbuf, acc, sidx, sbuf, drow,
                  semq, semk, semv):
    cidx = lax.axis_index("c")
    sidx_ax = lax.axis_index("s")
    wid = sidx_ax * 2 + cidx
    nbase = wid * NPW
    pltpu.sync_copy(srcp_hbm.at[wid], src_v)
    pltpu.sync_copy(degw_hbm.at[wid], deg_v)
    iota = lax.iota(jnp.int32, 16)
    zero16 = jnp.zeros((16,), jnp.float32)

    def node_body(n, eoff):
        nid = nbase + n
        base16 = (n // 16) * 16
        dvals = plsc.load_gather(deg_v, [base16 + iota])
        deg = jnp.sum(jnp.where(iota == (n - base16), dvals, 0))
        pltpu.async_copy(q_hbm.at[nid], qrow, semq).wait()
        for i in range(HC // 16):
            acc[pl.ds(i * 16, 16)] = zero16
        nch = (deg + 15) // 16

        def chunk_body(cc, den_vec):
            idx16 = eoff + cc * 16 + iota
            valid = idx16 < eoff + deg
            vals = plsc.load_gather(src_v, [jnp.minimum(idx16, ECAP - 1)])
            sidx[...] = jnp.where(valid, vals, 0)
            ck = pltpu.async_copy(k_hbm.at[sidx], kbuf, semk)
            cv = pltpu.async_copy(v_hbm.at[sidx], vbuf, semv)
            ck.wait()
            nleft = jnp.minimum(deg - cc * 16, 16)

            def alpha_body(e, dv):
                rowv = jnp.full((16,), e, jnp.int32)
                alv = zero16
                for h in range(H):
                    a = zero16
                    for i in range(16):
                        col = h * C + i * 16
                        a = a + qrow[pl.ds(col, 16)] * plsc.load_gather(
                            kbuf, [rowv, col + iota])
                    alv = alv + jnp.where(
                        iota == h, jnp.full((16,), jnp.sum(a)), 0.0)
                svec = jnp.exp(alv * INV_SQRT_C)
                plsc.store_scatter(
                    sbuf, [jnp.where(iota < 8, iota * 16 + e, 0)], svec,
                    mask=iota < 8)
                return dv + jnp.where(iota < 8, svec, 0.0)

            den_vec = lax.fori_loop(0, nleft, alpha_body, den_vec)
            cv.wait()

            def accum_body(e, carry):
                rowv = jnp.full((16,), e, jnp.int32)
                for h in range(H):
                    sb = plsc.load_gather(sbuf, [jnp.full((16,), h * 16, jnp.int32) + e])
                    for i in range(16):
                        col = h * C + i * 16
                        plsc.addupdate(
                            acc.at[pl.ds(col, 16)],
                            sb * plsc.load_gather(vbuf, [rowv, col + iota]))
                return carry

            lax.fori_loop(0, nleft, accum_body, 0)
            return den_vec

        den_vec = lax.fori_loop(0, nch, chunk_body, zero16)
        drow[...] = den_vec
        pltpu.sync_copy(acc, numer_hbm.at[nid])
        pltpu.sync_copy(drow, den_hbm.at[nid])
        return eoff + deg

    lax.fori_loop(0, NPW, node_body, jnp.int32(0))


def _edge_phase_sc(q, k, v, srcp, degw):
    mesh = plsc.VectorSubcoreMesh(core_axis_name="c", subcore_axis_name="s")
    f = pl.kernel(
        _edge_sc_body,
        out_type=[
            jax.ShapeDtypeStruct((N_PAD, HC), jnp.float32),
            jax.ShapeDtypeStruct((N_PAD, 16), jnp.float32),
        ],
        mesh=mesh,
        compiler_params=pltpu.CompilerParams(needs_layout_passes=False),
        scratch_types=[
            pltpu.VMEM((ECAP,), jnp.int32),
            pltpu.VMEM((NPW,), jnp.int32),
            pltpu.VMEM((HC,), jnp.float32),
            pltpu.VMEM((16, HC), jnp.float32),
            pltpu.VMEM((16, HC), jnp.float32),
            pltpu.VMEM((HC,), jnp.float32),
            pltpu.VMEM((16,), jnp.int32),
            pltpu.VMEM((128,), jnp.float32),
            pltpu.VMEM((16,), jnp.float32),
            pltpu.SemaphoreType.DMA,
            pltpu.SemaphoreType.DMA,
            pltpu.SemaphoreType.DMA,
        ],
    )
    return f(q, k, v, srcp, degw)


def kernel(x, edge_index, params):
    ids = x[:, 0, :].astype(jnp.int32)
    ids_pad = jnp.pad(ids, ((0, N_PAD - N_NODES), (0, 5)),
                      constant_values=NTAB - 1)
    emb_pad = jnp.zeros((NTAB, D), jnp.float32).at[:25].set(params["node_emb"])

    src = edge_index[0]
    dst = edge_index[1]
    order = jnp.argsort(dst)
    src_s = src[order].astype(jnp.int32)
    dst_s = dst[order].astype(jnp.int32)
    E = src_s.shape[0]
    rowptr = jnp.searchsorted(dst_s, jnp.arange(N_PAD + 1, dtype=jnp.int32),
                              side="left").astype(jnp.int32)
    deg = rowptr[1:] - rowptr[:-1]
    degw = deg.reshape(32, NPW)
    rp0 = rowptr[::NPW][:32]
    srcp = src_s[jnp.clip(rp0[:, None] + jnp.arange(ECAP, dtype=jnp.int32),
                          0, E - 1)]

    h = _embed(ids_pad, emb_pad)

    for lp in params["layers"]:
        wcat = jnp.concatenate(
            [lp["Wq"].T, lp["Wk"].T, lp["Wv"].T, lp["Wskip"].T], axis=1)
        bcat = jnp.zeros((8, 3 * HC + D), jnp.float32).at[0].set(
            jnp.concatenate([lp["bq"], lp["bk"], lp["bv"], lp["bskip"]]))
        gb1 = jnp.zeros((8, D), jnp.float32).at[0].set(lp["ln1_g"]).at[1].set(
            lp["ln1_b"])
        q, k, v, skip = _preattn(h, wcat, bcat, gb1)

        numer, den16 = _edge_phase_sc(q, k, v, srcp, degw)

        gb2 = jnp.zeros((8, D), jnp.float32).at[0].set(lp["ln2_g"]).at[1].set(
            lp["ln2_b"])
        bff = jnp.zeros((8, D), jnp.float32).at[0].set(lp["b1"]).at[1].set(
            lp["b2"])
        h = _post(h, numer, den16, skip, lp["W1"].T, lp["W2"].T, bff, gb2)

    wm = jnp.concatenate([w.T for (w, _) in params["mlp"]], axis=0)
    gbw = jnp.zeros((8, D), jnp.float32).at[0].set(params["fln_g"]).at[1].set(
        params["fln_b"]).at[2].set(params["Wout"][0])
    parts = _final(h, wm, gbw)
    total = jnp.sum(parts[::8, 0])
    return total / N_NODES + params["bout"][0]
